# Initial kernel scaffold; baseline (speedup 1.0000x reference)
import jax
import jax.numpy as jnp
from jax.experimental import pallas as pl


def kernel(x, table):
    raise NotImplementedError("write your pallas kernel here")


# speedup vs baseline: 23.4184x; 23.4184x over previous
import numpy as np
import jax
import jax.numpy as jnp
from jax import lax
from jax.experimental import pallas as pl
from jax.experimental.pallas import tpu as pltpu
from jax.experimental.pallas import tpu_sc as plsc

N_LEVELS = 16
F_PER_LEVEL = 2
LOG2_T = 19
T = 1 << LOG2_T
BASE_RES = 16
PLS = 1.5
PRIME_Y_I32 = np.int32(np.uint32(2654435761).view(np.int32))
MASK = T - 1

NC, NS, L = 2, 16, 16
NW = NC * NS
NPTS = 16 * 128 * 512
P = NPTS // NW
C = 512
NCH = P // C
VPC = C // L

_SCALES, _RES, _DENSE = [], [], []
for _l in range(N_LEVELS):
    _s = float(np.exp2(_l * np.log2(PLS)) * BASE_RES - 1.0)
    _r = int(np.ceil(_s)) + 1
    _SCALES.append(np.float32(_s))
    _RES.append(_r)
    _DENSE.append((_r ** 2) <= T)
N_DENSE = sum(_DENSE)

def _tile_body(x_hbm, tab_hbm, out_hbm,
               cbuf, pxb, pyb, wxb, wyb, idxb, offb, rows, obuf, sem):
    cid = lax.axis_index("c")
    sid = lax.axis_index("s")
    wid = sid * NC + cid

    iota = lax.iota(jnp.int32, L)
    iota2 = iota * 2
    zeros_i = iota * 0
    ones_i = zeros_i + 1

    @pl.loop(0, NCH)
    def _chunk(ch):
        pbase = wid * P + ch * C
        pltpu.sync_copy(x_hbm.at[pl.ds(pbase * 2, 2 * C)], cbuf)

        @pl.loop(0, VPC)
        def _pre(i):
            xi = plsc.load_gather(cbuf, [iota2 + 32 * i])
            yi = plsc.load_gather(cbuf, [iota2 + (32 * i + 1)])
            pxb[pl.ds(16 * i, 16)] = xi * 0.5 + 0.5
            pyb[pl.ds(16 * i, 16)] = yi * 0.5 + 0.5

        def _phase_a(i, l, hashed):
            scale = float(_SCALES[l])
            base = l * T
            px = pxb[pl.ds(16 * i, 16)]
            py = pyb[pl.ds(16 * i, 16)]
            fx = px * scale + 0.5
            fy = py * scale + 0.5
            ix = fx.astype(jnp.int32)
            iy = fy.astype(jnp.int32)
            hx = fx - ix.astype(jnp.float32)
            hy = fy - iy.astype(jnp.float32)
            wxb[pl.ds(16 * i, 16)] = hx * hx * (3.0 - 2.0 * hx)
            wyb[pl.ds(16 * i, 16)] = hy * hy * (3.0 - 2.0 * hy)
            if hashed:
                h0 = iy * PRIME_Y_I32
                h1 = (iy + 1) * PRIME_Y_I32
                i00 = ((ix ^ h0) & MASK) + base
                i01 = ((ix ^ h1) & MASK) + base
                i10 = (((ix + 1) ^ h0) & MASK) + base
                i11 = (((ix + 1) ^ h1) & MASK) + base
            else:
                res = _RES[l]
                lim = l * T + min(res ** 2, T) - 1
                r0 = iy * res + base
                r1 = r0 + res
                i00 = jnp.minimum(ix + r0, lim)
                i01 = jnp.minimum(ix + r1, lim)
                i10 = jnp.minimum(ix + 1 + r0, lim)
                i11 = jnp.minimum(ix + 1 + r1, lim)
            idxb[pl.ds(16 * i, 16)] = i00 >> 2
            idxb[pl.ds(C + 16 * i, 16)] = i01 >> 2
            idxb[pl.ds(2 * C + 16 * i, 16)] = i10 >> 2
            idxb[pl.ds(3 * C + 16 * i, 16)] = i11 >> 2
            offb[pl.ds(16 * i, 16)] = i00
            offb[pl.ds(C + 16 * i, 16)] = i01
            offb[pl.ds(2 * C + 16 * i, 16)] = i10
            offb[pl.ds(3 * C + 16 * i, 16)] = i11

        def _phase_b(i, l):
            wx = wxb[pl.ds(16 * i, 16)]
            wy = wyb[pl.ds(16 * i, 16)]
            u = 1.0 - wx
            v = 1.0 - wy
            w4 = (u * v, u * wy, wx * v, wx * wy)
            acc0 = None
            for c4 in range(4):
                ridx = iota + (c4 * C + 16 * i)
                off = (offb[pl.ds(c4 * C + 16 * i, 16)] & 3) << 1
                f0 = plsc.load_gather(rows, [ridx, off])
                f1 = plsc.load_gather(rows, [ridx, off + 1])
                if acc0 is None:
                    acc0 = f0 * w4[c4]
                    acc1 = f1 * w4[c4]
                else:
                    acc0 = acc0 + f0 * w4[c4]
                    acc1 = acc1 + f1 * w4[c4]
            pts = iota + 16 * i
            col = zeros_i + (2 * l)
            plsc.store_scatter(obuf, [pts, col], acc0)
            plsc.store_scatter(obuf, [pts, col + 1], acc1)

        def _level(l, hashed):
            @pl.loop(0, VPC)
            def _a(i):
                _phase_a(i, l, hashed)
            descs = []
            for seg in range(4 * C // 128):
                descs.append(pltpu.async_copy(
                    tab_hbm.at[idxb.at[pl.ds(seg * 128, 128)]],
                    rows.at[pl.ds(seg * 128, 128)], sem))
            for d in descs:
                d.wait()

            @pl.loop(0, VPC)
            def _b(i):
                _phase_b(i, l)

        for l in range(N_LEVELS):
            _level(l, hashed=not _DENSE[l])

        pltpu.sync_copy(obuf, out_hbm.at[pl.ds(pbase, C)])


def _make_kernel():
    mesh = plsc.VectorSubcoreMesh(core_axis_name="c", subcore_axis_name="s")
    return pl.kernel(
        _tile_body,
        out_type=jax.ShapeDtypeStruct((NPTS, 2 * N_LEVELS), jnp.float32),
        mesh=mesh,
        compiler_params=pltpu.CompilerParams(
            needs_layout_passes=False, use_tc_tiling_on_sc=False),
        scratch_types=[
            pltpu.VMEM((2 * C,), jnp.float32),
            pltpu.VMEM((C,), jnp.float32),
            pltpu.VMEM((C,), jnp.float32),
            pltpu.VMEM((C,), jnp.float32),
            pltpu.VMEM((C,), jnp.float32),
            pltpu.VMEM((4 * C,), jnp.int32),
            pltpu.VMEM((4 * C,), jnp.int32),
            pltpu.VMEM((4 * C, 8), jnp.float32),
            pltpu.VMEM((C, 2 * N_LEVELS), jnp.float32),
            pltpu.SemaphoreType.DMA,
        ],
    )


_KERNEL = _make_kernel()


def kernel(x, table):
    Nv, Nr, Nm, _ = x.shape
    xf = x.reshape(-1)
    tab = table.reshape(-1, 8)
    out = _KERNEL(xf, tab)
    return out.reshape(Nv, Nr, Nm, 2 * N_LEVELS)
